# baseline (speedup 1.0000x reference)
import functools

import jax
import jax.numpy as jnp
from jax import lax
from jax.experimental import pallas as pl
from jax.experimental.pallas import tpu as pltpu
from jax.experimental.pallas import tpu_sc as plsc

N = 10000
H = 128
E = 320000
NC = 2
NS = 16
CH = 128
NPAD = 10240
RPT16 = NPAD // NS
EPAD = 327680
DCH = EPAD // (NS * CH)
PB = 40
BR = 1024

_mesh = plsc.VectorSubcoreMesh(core_axis_name="c", subcore_axis_name="s")


@functools.partial(
    pl.kernel,
    out_type=jax.ShapeDtypeStruct((NC * NPAD,), jnp.float32),
    mesh=_mesh,
    scratch_types=[
        pltpu.VMEM((DCH // 4, CH), jnp.int32),
        pltpu.VMEM((CH,), jnp.float32),
        pltpu.VMEM_SHARED((NPAD,), jnp.float32),
    ],
)
def _sc_degree(dst_hbm, ones_hbm, zeros1_hbm, out_hbm, dst_v, ones_v, sdeg):
    c = lax.axis_index("c")
    s = lax.axis_index("s")
    pltpu.sync_copy(ones_hbm, ones_v)
    pltpu.sync_copy(zeros1_hbm, sdeg.at[pl.ds(s * RPT16, RPT16)])
    plsc.subcore_barrier()

    def body(j, _):
        pltpu.sync_copy(ones_v, sdeg.at[dst_v.at[j]], add=True)
        return ()

    for b in range(4):
        pltpu.sync_copy(dst_hbm.at[c, s, pl.ds(b * (DCH // 4), DCH // 4)],
                        dst_v)
        lax.fori_loop(0, DCH // 4, body, (), unroll=False)
    plsc.subcore_barrier()
    pltpu.sync_copy(sdeg.at[pl.ds(s * RPT16, RPT16)],
                    out_hbm.at[pl.ds(c * NPAD + s * RPT16, RPT16)])


@functools.partial(
    pl.kernel,
    out_type=jax.ShapeDtypeStruct((NC * NPAD, H), jnp.float32),
    mesh=_mesh,
    scratch_types=[
        pltpu.VMEM((PB, CH), jnp.int32),
        pltpu.VMEM((PB, CH), jnp.int32),
        pltpu.VMEM((CH, H), jnp.float32),
        pltpu.VMEM((CH, H), jnp.float32),
        pltpu.VMEM_SHARED((NPAD, H), jnp.float32),
        pltpu.SemaphoreType.DMA,
        pltpu.SemaphoreType.DMA,
        pltpu.SemaphoreType.DMA,
        pltpu.SemaphoreType.DMA,
    ],
)
def _sc_scatter(g_hbm, src_hbm, dst_hbm, zeros_hbm, out_hbm,
                src_v, dst_v, rows0, rows1, ush, gsem_a, gsem_b,
                ssem_a, ssem_b):
    c = lax.axis_index("c")
    s = lax.axis_index("s")
    pltpu.sync_copy(zeros_hbm, ush.at[pl.ds(s * RPT16, RPT16)])
    plsc.subcore_barrier()

    def gstart(j, buf, sm):
        for k in range(8):
            idxv = src_v[j, pl.ds(16 * k, 16)]
            pltpu.async_copy(g_hbm.at[idxv], buf.at[pl.ds(16 * k, 16)], sm)

    def gwait(buf, sm):
        for k in range(8):
            pltpu.make_async_copy(g_hbm.at[pl.ds(0, 16)],
                                  buf.at[pl.ds(16 * k, 16)], sm).wait()

    def sstart(j, buf, sm):
        pltpu.async_copy(buf, ush.at[dst_v.at[j]], sm, add=True)

    def swaitb(buf, sm):
        pltpu.make_async_copy(buf, ush.at[dst_v.at[0]], sm).wait()

    for b in range(DCH // PB):
        pltpu.sync_copy(src_hbm.at[c, s, pl.ds(b * PB, PB)], src_v)
        pltpu.sync_copy(dst_hbm.at[c, s, pl.ds(b * PB, PB)], dst_v)
        gstart(0, rows0, gsem_a)
        gstart(1, rows1, gsem_b)

        def body(p, _):
            j = 2 * p
            gwait(rows0, gsem_a)
            sstart(j, rows0, ssem_a)
            gwait(rows1, gsem_b)
            sstart(j + 1, rows1, ssem_b)
            swaitb(rows0, ssem_a)
            gstart(j + 2, rows0, gsem_a)
            swaitb(rows1, ssem_b)
            gstart(j + 3, rows1, gsem_b)
            return ()

        lax.fori_loop(0, PB // 2 - 1, body, (), unroll=False)
        gwait(rows0, gsem_a)
        sstart(PB - 2, rows0, ssem_a)
        gwait(rows1, gsem_b)
        sstart(PB - 1, rows1, ssem_b)
        swaitb(rows0, ssem_a)
        swaitb(rows1, ssem_b)
    plsc.subcore_barrier()
    pltpu.sync_copy(ush.at[pl.ds(s * RPT16, RPT16)],
                    out_hbm.at[pl.ds(c * NPAD + s * RPT16, RPT16)])


def _pre_body(h_ref, w_ref, deg_ref, g_ref):
    j = pl.program_id(0)
    i = j % (NPAD // BR)
    rows = i * BR + lax.broadcasted_iota(jnp.int32, (BR, 1), 0)
    mask = (rows < N).astype(jnp.float32)
    dis = lax.rsqrt(deg_ref[...] + 1.0) * mask
    g_ref[...] = jnp.dot(h_ref[...], w_ref[...],
                         preferred_element_type=jnp.float32) * dis


def _tc_pre(h, wcat, deg):
    nb = NPAD // BR
    return pl.pallas_call(
        _pre_body,
        grid=(2 * nb,),
        in_specs=[
            pl.BlockSpec((BR, H), lambda j: (j % nb, 0)),
            pl.BlockSpec((H, H), lambda j: (0, j // nb)),
            pl.BlockSpec((BR, 1), lambda j: (j, 0)),
        ],
        out_specs=pl.BlockSpec((BR, H), lambda j: (j, 0)),
        out_shape=jax.ShapeDtypeStruct((2 * NPAD, H), jnp.float32),
    )(h, wcat, deg)


def _postpre_body(u_ref, g_ref, deg_ref, h_ref, wih_ref, whh_ref,
                  bih_ref, bhh_ref, ba_ref, bc_ref, wcat_ref,
                  hout_ref, gout_ref):
    i = pl.program_id(0)
    rows = i * BR + lax.broadcasted_iota(jnp.int32, (BR, 1), 0)
    mask = (rows < N).astype(jnp.float32)
    dis_a = lax.rsqrt(deg_ref[0] + 1.0) * mask
    dis_c = lax.rsqrt(deg_ref[1] + 1.0) * mask
    a = ((u_ref[0] + g_ref[0]) * dis_a + ba_ref[...]
         + (u_ref[1] + g_ref[1]) * dis_c + bc_ref[...])
    h = h_ref[...]
    gi = jnp.dot(a, wih_ref[...], preferred_element_type=jnp.float32) + bih_ref[...]
    gh = jnp.dot(h, whh_ref[...], preferred_element_type=jnp.float32) + bhh_ref[...]
    r = jax.nn.sigmoid(gi[:, :H] + gh[:, :H])
    z = jax.nn.sigmoid(gi[:, H:2 * H] + gh[:, H:2 * H])
    n = jnp.tanh(gi[:, 2 * H:] + r * gh[:, 2 * H:])
    hn = (1.0 - z) * n + z * h
    hout_ref[...] = hn
    hw = jnp.dot(hn, wcat_ref[...], preferred_element_type=jnp.float32)
    gout_ref[0] = hw[:, :H] * dis_a
    gout_ref[1] = hw[:, H:] * dis_c


def _tc_postpre(u3, g3, deg3, h, wiht, whht, bih, bhh, ba, bc, wcat):
    blk = lambda i: (i, 0)
    return pl.pallas_call(
        _postpre_body,
        grid=(NPAD // BR,),
        in_specs=[
            pl.BlockSpec((2, BR, H), lambda i: (0, i, 0)),
            pl.BlockSpec((2, BR, H), lambda i: (0, i, 0)),
            pl.BlockSpec((2, BR, 1), lambda i: (0, i, 0)),
            pl.BlockSpec((BR, H), blk),
            pl.BlockSpec((H, 3 * H), lambda i: (0, 0)),
            pl.BlockSpec((H, 3 * H), lambda i: (0, 0)),
            pl.BlockSpec((1, 3 * H), lambda i: (0, 0)),
            pl.BlockSpec((1, 3 * H), lambda i: (0, 0)),
            pl.BlockSpec((1, H), lambda i: (0, 0)),
            pl.BlockSpec((1, H), lambda i: (0, 0)),
            pl.BlockSpec((H, 2 * H), lambda i: (0, 0)),
        ],
        out_specs=[
            pl.BlockSpec((BR, H), blk),
            pl.BlockSpec((2, BR, H), lambda i: (0, i, 0)),
        ],
        out_shape=[
            jax.ShapeDtypeStruct((NPAD, H), jnp.float32),
            jax.ShapeDtypeStruct((2, NPAD, H), jnp.float32),
        ],
    )(u3, g3, deg3, h, wiht, whht, bih, bhh, ba, bc, wcat)


def kernel(x, edge_ast, edge_cfg, W_ast, b_ast, W_cfg, b_cfg,
           W_ih, W_hh, b_ih, b_hh):
    pad = jnp.full((EPAD - E,), N, dtype=jnp.int32)
    src_a = jnp.concatenate([edge_ast[0].astype(jnp.int32), pad])
    dst_a = jnp.concatenate([edge_ast[1].astype(jnp.int32), pad])
    src_c = jnp.concatenate([edge_cfg[0].astype(jnp.int32), pad]) + NPAD
    dst_c = jnp.concatenate([edge_cfg[1].astype(jnp.int32), pad])

    dst4 = jnp.stack([dst_a.reshape(NS, DCH, CH), dst_c.reshape(NS, DCH, CH)])
    src4 = jnp.stack([src_a.reshape(NS, DCH, CH), src_c.reshape(NS, DCH, CH)])
    ones = jnp.ones((CH,), jnp.float32)
    zeros1 = jnp.zeros((RPT16,), jnp.float32)
    zeros2 = jnp.zeros((RPT16, H), jnp.float32)

    deg2 = _sc_degree(dst4, ones, zeros1)
    deg = deg2.reshape(2 * NPAD, 1)
    deg3 = deg2.reshape(2, NPAD, 1)

    h = jnp.pad(x, ((0, NPAD - N), (0, 0)))
    wcat = jnp.concatenate([W_ast, W_cfg], axis=1)
    wiht = W_ih.T
    whht = W_hh.T
    bih = b_ih.reshape(1, 3 * H)
    bhh = b_hh.reshape(1, 3 * H)
    ba = b_ast.reshape(1, H)
    bc = b_cfg.reshape(1, H)

    g = _tc_pre(h, wcat, deg)
    for _ in range(3):
        u = _sc_scatter(g, src4, dst4, zeros2)
        h, g3 = _tc_postpre(u.reshape(2, NPAD, H), g.reshape(2, NPAD, H),
                            deg3, h, wiht, whht, bih, bhh, ba, bc, wcat)
        g = g3.reshape(2 * NPAD, H)
    return h[:N]


# speedup vs baseline: 1.0889x; 1.0889x over previous
import functools

import jax
import jax.numpy as jnp
from jax import lax
from jax.experimental import pallas as pl
from jax.experimental.pallas import tpu as pltpu
from jax.experimental.pallas import tpu_sc as plsc

N = 10000
H = 128
E = 320000
NC = 2
NS = 16
CH = 128
NPAD = 10240
RPT16 = NPAD // NS
EPAD = 327680
DCH = EPAD // (NS * CH)
PB = 40
BR = 1024

_mesh = plsc.VectorSubcoreMesh(core_axis_name="c", subcore_axis_name="s")


@functools.partial(
    pl.kernel,
    out_type=jax.ShapeDtypeStruct((NC * NPAD,), jnp.float32),
    mesh=_mesh,
    scratch_types=[
        pltpu.VMEM((DCH // 4, CH), jnp.int32),
        pltpu.VMEM((CH,), jnp.float32),
        pltpu.VMEM_SHARED((NPAD,), jnp.float32),
    ],
)
def _sc_degree(dst_hbm, ones_hbm, zeros1_hbm, out_hbm, dst_v, ones_v, sdeg):
    c = lax.axis_index("c")
    s = lax.axis_index("s")
    pltpu.sync_copy(ones_hbm, ones_v)
    pltpu.sync_copy(zeros1_hbm, sdeg.at[pl.ds(s * RPT16, RPT16)])
    plsc.subcore_barrier()

    def body(j, _):
        pltpu.sync_copy(ones_v, sdeg.at[dst_v.at[j]], add=True)
        return ()

    for b in range(4):
        pltpu.sync_copy(dst_hbm.at[c, s, pl.ds(b * (DCH // 4), DCH // 4)],
                        dst_v)
        lax.fori_loop(0, DCH // 4, body, (), unroll=False)
    plsc.subcore_barrier()
    pltpu.sync_copy(sdeg.at[pl.ds(s * RPT16, RPT16)],
                    out_hbm.at[pl.ds(c * NPAD + s * RPT16, RPT16)])


@functools.partial(
    pl.kernel,
    out_type=jax.ShapeDtypeStruct((NC * NPAD, H), jnp.float32),
    mesh=_mesh,
    scratch_types=[
        pltpu.VMEM((PB, CH), jnp.int32),
        pltpu.VMEM((PB, CH), jnp.int32),
        pltpu.VMEM((CH, H), jnp.float32),
        pltpu.VMEM((CH, H), jnp.float32),
        pltpu.VMEM_SHARED((NPAD, H), jnp.float32),
        pltpu.SemaphoreType.DMA,
        pltpu.SemaphoreType.DMA,
    ],
)
def _sc_scatter(g_hbm, src_hbm, dst_hbm, zeros_hbm, out_hbm,
                src_v, dst_v, rows0, rows1, ush, gsem_a, gsem_b):
    c = lax.axis_index("c")
    s = lax.axis_index("s")
    pltpu.sync_copy(zeros_hbm, ush.at[pl.ds(s * RPT16, RPT16)])
    plsc.subcore_barrier()

    def gstart(j, buf, sm):
        for k in range(8):
            idxv = src_v[j, pl.ds(16 * k, 16)]
            pltpu.async_copy(g_hbm.at[idxv], buf.at[pl.ds(16 * k, 16)], sm)

    def gwait(buf, sm):
        for k in range(8):
            pltpu.make_async_copy(g_hbm.at[pl.ds(0, 16)],
                                  buf.at[pl.ds(16 * k, 16)], sm).wait()

    def scat(j, buf):
        pltpu.sync_copy(buf, ush.at[dst_v.at[j]], add=True)

    for b in range(DCH // PB):
        pltpu.sync_copy(src_hbm.at[c, s, pl.ds(b * PB, PB)], src_v)
        pltpu.sync_copy(dst_hbm.at[c, s, pl.ds(b * PB, PB)], dst_v)
        gstart(0, rows0, gsem_a)
        gstart(1, rows1, gsem_b)

        def body(p, _):
            j = 2 * p
            gwait(rows0, gsem_a)
            scat(j, rows0)
            gstart(j + 2, rows0, gsem_a)
            gwait(rows1, gsem_b)
            scat(j + 1, rows1)
            gstart(j + 3, rows1, gsem_b)
            return ()

        lax.fori_loop(0, PB // 2 - 1, body, (), unroll=False)
        gwait(rows0, gsem_a)
        scat(PB - 2, rows0)
        gwait(rows1, gsem_b)
        scat(PB - 1, rows1)
    plsc.subcore_barrier()
    pltpu.sync_copy(ush.at[pl.ds(s * RPT16, RPT16)],
                    out_hbm.at[pl.ds(c * NPAD + s * RPT16, RPT16)])


def _pre_body(h_ref, w_ref, deg_ref, g_ref):
    j = pl.program_id(0)
    i = j % (NPAD // BR)
    rows = i * BR + lax.broadcasted_iota(jnp.int32, (BR, 1), 0)
    mask = (rows < N).astype(jnp.float32)
    dis = lax.rsqrt(deg_ref[...] + 1.0) * mask
    g_ref[...] = jnp.dot(h_ref[...], w_ref[...],
                         preferred_element_type=jnp.float32) * dis


def _tc_pre(h, wcat, deg):
    nb = NPAD // BR
    return pl.pallas_call(
        _pre_body,
        grid=(2 * nb,),
        in_specs=[
            pl.BlockSpec((BR, H), lambda j: (j % nb, 0)),
            pl.BlockSpec((H, H), lambda j: (0, j // nb)),
            pl.BlockSpec((BR, 1), lambda j: (j, 0)),
        ],
        out_specs=pl.BlockSpec((BR, H), lambda j: (j, 0)),
        out_shape=jax.ShapeDtypeStruct((2 * NPAD, H), jnp.float32),
    )(h, wcat, deg)


def _postpre_body(u_ref, g_ref, deg_ref, h_ref, wih_ref, whh_ref,
                  bih_ref, bhh_ref, ba_ref, bc_ref, wcat_ref,
                  hout_ref, gout_ref):
    i = pl.program_id(0)
    rows = i * BR + lax.broadcasted_iota(jnp.int32, (BR, 1), 0)
    mask = (rows < N).astype(jnp.float32)
    dis_a = lax.rsqrt(deg_ref[0] + 1.0) * mask
    dis_c = lax.rsqrt(deg_ref[1] + 1.0) * mask
    a = ((u_ref[0] + g_ref[0]) * dis_a + ba_ref[...]
         + (u_ref[1] + g_ref[1]) * dis_c + bc_ref[...])
    h = h_ref[...]
    gi = jnp.dot(a, wih_ref[...], preferred_element_type=jnp.float32) + bih_ref[...]
    gh = jnp.dot(h, whh_ref[...], preferred_element_type=jnp.float32) + bhh_ref[...]
    r = jax.nn.sigmoid(gi[:, :H] + gh[:, :H])
    z = jax.nn.sigmoid(gi[:, H:2 * H] + gh[:, H:2 * H])
    n = jnp.tanh(gi[:, 2 * H:] + r * gh[:, 2 * H:])
    hn = (1.0 - z) * n + z * h
    hout_ref[...] = hn
    hw = jnp.dot(hn, wcat_ref[...], preferred_element_type=jnp.float32)
    gout_ref[0] = hw[:, :H] * dis_a
    gout_ref[1] = hw[:, H:] * dis_c


def _tc_postpre(u3, g3, deg3, h, wiht, whht, bih, bhh, ba, bc, wcat):
    blk = lambda i: (i, 0)
    return pl.pallas_call(
        _postpre_body,
        grid=(NPAD // BR,),
        in_specs=[
            pl.BlockSpec((2, BR, H), lambda i: (0, i, 0)),
            pl.BlockSpec((2, BR, H), lambda i: (0, i, 0)),
            pl.BlockSpec((2, BR, 1), lambda i: (0, i, 0)),
            pl.BlockSpec((BR, H), blk),
            pl.BlockSpec((H, 3 * H), lambda i: (0, 0)),
            pl.BlockSpec((H, 3 * H), lambda i: (0, 0)),
            pl.BlockSpec((1, 3 * H), lambda i: (0, 0)),
            pl.BlockSpec((1, 3 * H), lambda i: (0, 0)),
            pl.BlockSpec((1, H), lambda i: (0, 0)),
            pl.BlockSpec((1, H), lambda i: (0, 0)),
            pl.BlockSpec((H, 2 * H), lambda i: (0, 0)),
        ],
        out_specs=[
            pl.BlockSpec((BR, H), blk),
            pl.BlockSpec((2, BR, H), lambda i: (0, i, 0)),
        ],
        out_shape=[
            jax.ShapeDtypeStruct((NPAD, H), jnp.float32),
            jax.ShapeDtypeStruct((2, NPAD, H), jnp.float32),
        ],
    )(u3, g3, deg3, h, wiht, whht, bih, bhh, ba, bc, wcat)


def kernel(x, edge_ast, edge_cfg, W_ast, b_ast, W_cfg, b_cfg,
           W_ih, W_hh, b_ih, b_hh):
    pad = jnp.full((EPAD - E,), N, dtype=jnp.int32)
    src_a = jnp.concatenate([edge_ast[0].astype(jnp.int32), pad])
    dst_a = jnp.concatenate([edge_ast[1].astype(jnp.int32), pad])
    src_c = jnp.concatenate([edge_cfg[0].astype(jnp.int32), pad]) + NPAD
    dst_c = jnp.concatenate([edge_cfg[1].astype(jnp.int32), pad])

    dst4 = jnp.stack([dst_a.reshape(NS, DCH, CH), dst_c.reshape(NS, DCH, CH)])
    src4 = jnp.stack([src_a.reshape(NS, DCH, CH), src_c.reshape(NS, DCH, CH)])
    ones = jnp.ones((CH,), jnp.float32)
    zeros1 = jnp.zeros((RPT16,), jnp.float32)
    zeros2 = jnp.zeros((RPT16, H), jnp.float32)

    deg2 = _sc_degree(dst4, ones, zeros1)
    deg = deg2.reshape(2 * NPAD, 1)
    deg3 = deg2.reshape(2, NPAD, 1)

    h = jnp.pad(x, ((0, NPAD - N), (0, 0)))
    wcat = jnp.concatenate([W_ast, W_cfg], axis=1)
    wiht = W_ih.T
    whht = W_hh.T
    bih = b_ih.reshape(1, 3 * H)
    bhh = b_hh.reshape(1, 3 * H)
    ba = b_ast.reshape(1, H)
    bc = b_cfg.reshape(1, H)

    g = _tc_pre(h, wcat, deg)
    for _ in range(3):
        u = _sc_scatter(g, src4, dst4, zeros2)
        h, g3 = _tc_postpre(u.reshape(2, NPAD, H), g.reshape(2, NPAD, H),
                            deg3, h, wiht, whht, bih, bhh, ba, bc, wcat)
        g = g3.reshape(2 * NPAD, H)
    return h[:N]
